# Initial kernel scaffold; baseline (speedup 1.0000x reference)
import jax
import jax.numpy as jnp
from jax.experimental import pallas as pl


def kernel(features, tables):
    raise NotImplementedError("write your pallas kernel here")


# speedup vs baseline: 1.7688x; 1.7688x over previous
import functools

import jax
import jax.numpy as jnp
from jax import lax
from jax.experimental import pallas as pl
from jax.experimental.pallas import tpu as pltpu
from jax.experimental.pallas import tpu_sc as plsc

_NUM_FEATURES = 26
_VOCAB = 100000
_EMBED_DIM = 16
_BATCH = 1024
_SEQ = 50

_R = _BATCH * _SEQ * _NUM_FEATURES
_CHUNK = 128
_N_CHUNKS = _R // _CHUNK
_NC = 2
_NS = 16
_NW = _NC * _NS
_CPW = _N_CHUNKS // _NW
_K = 13
_NG = _CPW // _K
_GROUP_ROWS = _K * _CHUNK
_ROWS_PER_W = _CPW * _CHUNK


@functools.lru_cache(maxsize=None)
def _make_kernel():
    mesh = plsc.VectorSubcoreMesh(core_axis_name="c", subcore_axis_name="s")

    @functools.partial(
        pl.kernel,
        mesh=mesh,
        out_type=jax.ShapeDtypeStruct((_R, _EMBED_DIM), jnp.float32),
        scratch_types=[
            pltpu.VMEM((_ROWS_PER_W,), jnp.int32),
            pltpu.VMEM((_GROUP_ROWS, _EMBED_DIM), jnp.float32),
            pltpu.SemaphoreType.DMA,
        ],
        compiler_params=pltpu.CompilerParams(use_tc_tiling_on_sc=False),
    )
    def k(tab_hbm, idx_hbm, out_hbm, idx_v, rows_v, gsem):
        wid = lax.axis_index("s") * _NC + lax.axis_index("c")
        r0 = wid * _ROWS_PER_W
        pltpu.sync_copy(idx_hbm.at[pl.ds(r0, _ROWS_PER_W)], idx_v)

        def group(g, carry):
            copies = [
                pltpu.async_copy(
                    tab_hbm.at[idx_v.at[pl.ds((g * _K + c) * _CHUNK, _CHUNK)]],
                    rows_v.at[pl.ds(c * _CHUNK, _CHUNK)],
                    gsem,
                )
                for c in range(_K)
            ]
            for cp in copies:
                cp.wait()
            pltpu.sync_copy(
                rows_v, out_hbm.at[pl.ds(r0 + g * _GROUP_ROWS, _GROUP_ROWS)]
            )
            return carry

        lax.fori_loop(0, _NG, group, 0)

    return k


def kernel(features, tables):
    offs = jnp.arange(_NUM_FEATURES, dtype=jnp.int32) * _VOCAB
    flat_idx = (
        features.reshape(_BATCH * _SEQ, _NUM_FEATURES) + offs
    ).reshape(_R)
    tab = tables.reshape(_NUM_FEATURES * _VOCAB, _EMBED_DIM)
    out = _make_kernel()(tab, flat_idx)
    return out.reshape(_BATCH, _SEQ, _NUM_FEATURES * _EMBED_DIM)


# speedup vs baseline: 2.6805x; 1.5154x over previous
import functools

import jax
import jax.numpy as jnp
from jax import lax
from jax.experimental import pallas as pl
from jax.experimental.pallas import tpu as pltpu
from jax.experimental.pallas import tpu_sc as plsc

_NUM_FEATURES = 26
_VOCAB = 100000
_EMBED_DIM = 16
_BATCH = 1024
_SEQ = 50

_NC = 2
_NS = 16
_NW = _NC * _NS
_NTASK = _NUM_FEATURES * _EMBED_DIM
_TPW = _NTASK // _NW
_SCHUNK = 10
_NSC = _SEQ // _SCHUNK
_GPR = _BATCH // 16


@functools.lru_cache(maxsize=None)
def _make_kernel():
    mesh = plsc.VectorSubcoreMesh(core_axis_name="c", subcore_axis_name="s")

    @functools.partial(
        pl.kernel,
        mesh=mesh,
        out_type=jax.ShapeDtypeStruct(
            (_SEQ, _NUM_FEATURES * _EMBED_DIM, _BATCH), jnp.float32
        ),
        scratch_types=[
            pltpu.VMEM((_VOCAB,), jnp.float32),
            pltpu.VMEM((_SCHUNK, _BATCH), jnp.int32),
            pltpu.VMEM((_SCHUNK, _BATCH), jnp.float32),
        ],
        compiler_params=pltpu.CompilerParams(
            use_tc_tiling_on_sc=False, needs_layout_passes=False
        ),
    )
    def k(tab_hbm, feat_hbm, out_hbm, slab_v, idx_v, outb_v):
        wid = lax.axis_index("s") * _NC + lax.axis_index("c")

        def task(j, carry):
            t = wid * _TPW + j
            i = t // _EMBED_DIM
            d = t % _EMBED_DIM
            c = i * _EMBED_DIM + d
            pltpu.sync_copy(tab_hbm.at[i, d], slab_v)

            def schunk(sc, carry2):
                s0 = sc * _SCHUNK
                pltpu.sync_copy(feat_hbm.at[i, pl.ds(s0, _SCHUNK)], idx_v)

                def srow(sl, carry3):
                    def grp(g, carry4):
                        for kk in range(8):
                            off = g * 128 + kk * 16
                            iv = idx_v[sl, pl.ds(off, 16)]
                            vals = plsc.load_gather(slab_v, [iv])
                            outb_v[sl, pl.ds(off, 16)] = vals
                        return carry4

                    lax.fori_loop(0, _GPR // 8, grp, 0)
                    return carry3

                lax.fori_loop(0, _SCHUNK, srow, 0)
                pltpu.sync_copy(outb_v, out_hbm.at[pl.ds(s0, _SCHUNK), c])
                return carry2

            lax.fori_loop(0, _NSC, schunk, 0)
            return carry

        lax.fori_loop(0, _TPW, task, 0)

    return k


def kernel(features, tables):
    feat_t = jnp.transpose(features, (2, 1, 0))
    tab_t = jnp.transpose(tables, (0, 2, 1))
    out_t = _make_kernel()(tab_t, feat_t)
    return jnp.transpose(out_t, (2, 0, 1))


# speedup vs baseline: 2.9990x; 1.1188x over previous
import functools

import jax
import jax.numpy as jnp
from jax import lax
from jax.experimental import pallas as pl
from jax.experimental.pallas import tpu as pltpu
from jax.experimental.pallas import tpu_sc as plsc

_NUM_FEATURES = 26
_VOCAB = 100000
_EMBED_DIM = 16
_BATCH = 1024
_SEQ = 50

_NC = 2
_NS = 16
_NW = _NC * _NS
_NTASK = _NUM_FEATURES * _EMBED_DIM
_TPW = _NTASK // _NW
_SCHUNK = 5
_NCHUNK = _SEQ // _SCHUNK


@functools.lru_cache(maxsize=None)
def _make_kernel():
    mesh = plsc.VectorSubcoreMesh(core_axis_name="c", subcore_axis_name="s")

    @functools.partial(
        pl.kernel,
        mesh=mesh,
        out_type=jax.ShapeDtypeStruct(
            (_SEQ, _NUM_FEATURES * _EMBED_DIM, _BATCH), jnp.float32
        ),
        scratch_types=[
            pltpu.VMEM((_VOCAB,), jnp.float32),
            pltpu.VMEM((2, _SCHUNK, _BATCH), jnp.int32),
            pltpu.VMEM((2, _SCHUNK, _BATCH), jnp.float32),
            pltpu.SemaphoreType.DMA,
            pltpu.SemaphoreType.DMA,
            pltpu.SemaphoreType.DMA,
        ],
        compiler_params=pltpu.CompilerParams(
            use_tc_tiling_on_sc=False, needs_layout_passes=False
        ),
    )
    def k(tab_hbm, feat_hbm, out_hbm, slab_v, idx_v, outb_v, slab_sem,
          idx_sem, out_sem):
        wid = lax.axis_index("s") * _NC + lax.axis_index("c")

        def task(j, carry):
            t = wid * _TPW + j
            i = t // _EMBED_DIM
            d = t % _EMBED_DIM
            c = i * _EMBED_DIM + d
            slab_cp = pltpu.async_copy(tab_hbm.at[i, d], slab_v, slab_sem)
            idx_cps = [None] * _NCHUNK
            out_cps = [None] * _NCHUNK
            for ch in (0, 1):
                idx_cps[ch] = pltpu.async_copy(
                    feat_hbm.at[i, pl.ds(ch * _SCHUNK, _SCHUNK)],
                    idx_v.at[ch], idx_sem,
                )
            slab_cp.wait()

            for ch in range(_NCHUNK):
                b = ch % 2
                idx_cps[ch].wait()
                if out_cps[ch - 2] is not None:
                    out_cps[ch - 2].wait()

                def srow(sl, carry3, _b=b):
                    def grp(g, carry4):
                        for kk in range(8):
                            off = g * 128 + kk * 16
                            iv = idx_v.at[_b][sl, pl.ds(off, 16)]
                            vals = plsc.load_gather(slab_v, [iv])
                            outb_v.at[_b][sl, pl.ds(off, 16)] = vals
                        return carry4

                    lax.fori_loop(0, _BATCH // 128, grp, 0)
                    return carry3

                lax.fori_loop(0, _SCHUNK, srow, 0)
                out_cps[ch] = pltpu.async_copy(
                    outb_v.at[b],
                    out_hbm.at[pl.ds(ch * _SCHUNK, _SCHUNK), c], out_sem,
                )
                if ch + 2 < _NCHUNK:
                    idx_cps[ch + 2] = pltpu.async_copy(
                        feat_hbm.at[i, pl.ds((ch + 2) * _SCHUNK, _SCHUNK)],
                        idx_v.at[b], idx_sem,
                    )
            out_cps[_NCHUNK - 2].wait()
            out_cps[_NCHUNK - 1].wait()
            return carry

        lax.fori_loop(0, _TPW, task, 0)

    return k


def kernel(features, tables):
    feat_t = jnp.transpose(features, (2, 1, 0))
    tab_t = jnp.transpose(tables, (0, 2, 1))
    out_t = _make_kernel()(tab_t, feat_t)
    return jnp.transpose(out_t, (2, 0, 1))


# speedup vs baseline: 4.4901x; 1.4972x over previous
import functools

import jax
import jax.numpy as jnp
from jax import lax
from jax.experimental import pallas as pl
from jax.experimental.pallas import tpu as pltpu
from jax.experimental.pallas import tpu_sc as plsc

_NUM_FEATURES = 26
_VOCAB = 100000
_EMBED_DIM = 16
_BATCH = 1024
_SEQ = 50

_NC = 2
_NS = 16
_NW = _NC * _NS
_NTASK = _NUM_FEATURES * _EMBED_DIM
_TPW = _NTASK // _NW
_SCHUNK = 5
_NCHUNK = _SEQ // _SCHUNK


@functools.lru_cache(maxsize=None)
def _make_kernel():
    mesh = plsc.VectorSubcoreMesh(core_axis_name="c", subcore_axis_name="s")

    @functools.partial(
        pl.kernel,
        mesh=mesh,
        out_type=jax.ShapeDtypeStruct(
            (_SEQ, _NUM_FEATURES * _EMBED_DIM, _BATCH), jnp.float32
        ),
        scratch_types=[
            pltpu.VMEM((_VOCAB,), jnp.float32),
            pltpu.VMEM((2, _SCHUNK, _BATCH), jnp.int32),
            pltpu.VMEM((2, _SCHUNK, _BATCH), jnp.float32),
            pltpu.SemaphoreType.DMA,
            pltpu.SemaphoreType.DMA,
            pltpu.SemaphoreType.DMA,
        ],
        compiler_params=pltpu.CompilerParams(
            use_tc_tiling_on_sc=False, needs_layout_passes=False
        ),
    )
    def k(tab_hbm, feat_hbm, out_hbm, slab_v, idx_v, outb_v, slab_sem,
          idx_sem, out_sem):
        wid = lax.axis_index("s") * _NC + lax.axis_index("c")

        def task(j, carry):
            t = wid * _TPW + j
            i = t // _EMBED_DIM
            d = t % _EMBED_DIM
            c = i * _EMBED_DIM + d
            slab_cp = pltpu.async_copy(tab_hbm.at[i, d], slab_v, slab_sem)
            idx_cps = [None] * _NCHUNK
            out_cps = [None] * _NCHUNK
            for ch in (0, 1):
                idx_cps[ch] = pltpu.async_copy(
                    feat_hbm.at[i, pl.ds(ch * _SCHUNK, _SCHUNK)],
                    idx_v.at[ch], idx_sem,
                )
            slab_cp.wait()

            for ch in range(_NCHUNK):
                b = ch % 2
                idx_cps[ch].wait()
                if out_cps[ch - 2] is not None:
                    out_cps[ch - 2].wait()

                def srow(sl, carry3, _b=b):
                    @plsc.parallel_loop(0, _BATCH // 128, 1, unroll=2)
                    def grp(g):
                        for kk in range(8):
                            off = g * 128 + kk * 16
                            iv = idx_v.at[_b][sl, pl.ds(off, 16)]
                            vals = plsc.load_gather(slab_v, [iv])
                            outb_v.at[_b][sl, pl.ds(off, 16)] = vals

                    return carry3

                lax.fori_loop(0, _SCHUNK, srow, 0)
                out_cps[ch] = pltpu.async_copy(
                    outb_v.at[b],
                    out_hbm.at[pl.ds(ch * _SCHUNK, _SCHUNK), c], out_sem,
                )
                if ch + 2 < _NCHUNK:
                    idx_cps[ch + 2] = pltpu.async_copy(
                        feat_hbm.at[i, pl.ds((ch + 2) * _SCHUNK, _SCHUNK)],
                        idx_v.at[b], idx_sem,
                    )
            out_cps[_NCHUNK - 2].wait()
            out_cps[_NCHUNK - 1].wait()
            return carry

        lax.fori_loop(0, _TPW, task, 0)

    return k


def kernel(features, tables):
    feat_t = jnp.transpose(features, (2, 1, 0))
    tab_t = jnp.transpose(tables, (0, 2, 1))
    out_t = _make_kernel()(tab_t, feat_t)
    return jnp.transpose(out_t, (2, 0, 1))


# speedup vs baseline: 5.2918x; 1.1786x over previous
import functools

import jax
import jax.numpy as jnp
from jax import lax
from jax.experimental import pallas as pl
from jax.experimental.pallas import tpu as pltpu
from jax.experimental.pallas import tpu_sc as plsc

_NUM_FEATURES = 26
_VOCAB = 100000
_EMBED_DIM = 16
_BATCH = 1024
_SEQ = 50

_NC = 2
_NS = 16
_NW = _NC * _NS
_NTASK = _NUM_FEATURES * _EMBED_DIM
_TPW = _NTASK // _NW
_SCHUNK = 5
_NCHUNK = _SEQ // _SCHUNK


@functools.lru_cache(maxsize=None)
def _make_kernel():
    mesh = plsc.VectorSubcoreMesh(core_axis_name="c", subcore_axis_name="s")

    @functools.partial(
        pl.kernel,
        mesh=mesh,
        out_type=jax.ShapeDtypeStruct(
            (_SEQ, _NUM_FEATURES * _EMBED_DIM // 8, _BATCH // 128, 8, 128),
            jnp.float32,
        ),
        scratch_types=[
            pltpu.VMEM((_VOCAB,), jnp.float32),
            pltpu.VMEM((2, _SCHUNK, _BATCH), jnp.int32),
            pltpu.VMEM((2, _SCHUNK, _BATCH // 128, 128), jnp.float32),
            pltpu.SemaphoreType.DMA,
            pltpu.SemaphoreType.DMA,
            pltpu.SemaphoreType.DMA,
        ],
        compiler_params=pltpu.CompilerParams(
            use_tc_tiling_on_sc=False, needs_layout_passes=False
        ),
    )
    def k(tab_hbm, feat_hbm, out_hbm, slab_v, idx_v, outb_v, slab_sem,
          idx_sem, out_sem):
        wid = lax.axis_index("s") * _NC + lax.axis_index("c")

        def task(j, carry):
            t = wid * _TPW + j
            i = t // _EMBED_DIM
            d = t % _EMBED_DIM
            c = i * _EMBED_DIM + d
            slab_cp = pltpu.async_copy(tab_hbm.at[i, d], slab_v, slab_sem)
            idx_cps = [None] * _NCHUNK
            out_cps = [None] * _NCHUNK
            for ch in (0, 1):
                idx_cps[ch] = pltpu.async_copy(
                    feat_hbm.at[i, pl.ds(ch * _SCHUNK, _SCHUNK)],
                    idx_v.at[ch], idx_sem,
                )
            slab_cp.wait()

            for ch in range(_NCHUNK):
                b = ch % 2
                idx_cps[ch].wait()
                if out_cps[ch - 2] is not None:
                    out_cps[ch - 2].wait()

                def srow(sl, carry3, _b=b):
                    @plsc.parallel_loop(0, _BATCH // 128, 1, unroll=4)
                    def grp(g):
                        for kk in range(8):
                            off = g * 128 + kk * 16
                            iv = idx_v.at[_b][sl, pl.ds(off, 16)]
                            vals = plsc.load_gather(slab_v, [iv])
                            outb_v.at[_b][sl, g, pl.ds(kk * 16, 16)] = vals

                    return carry3

                lax.fori_loop(0, _SCHUNK, srow, 0)
                out_cps[ch] = pltpu.async_copy(
                    outb_v.at[b],
                    out_hbm.at[pl.ds(ch * _SCHUNK, _SCHUNK), c // 8, :, c % 8],
                    out_sem,
                )
                if ch + 2 < _NCHUNK:
                    idx_cps[ch + 2] = pltpu.async_copy(
                        feat_hbm.at[i, pl.ds((ch + 2) * _SCHUNK, _SCHUNK)],
                        idx_v.at[b], idx_sem,
                    )
            out_cps[_NCHUNK - 2].wait()
            out_cps[_NCHUNK - 1].wait()
            return carry

        lax.fori_loop(0, _TPW, task, 0)

    return k


def kernel(features, tables):
    feat_t = jnp.transpose(features, (2, 1, 0))
    tab_t = jnp.transpose(tables, (0, 2, 1))
    out5 = _make_kernel()(tab_t, feat_t)
    out = jnp.transpose(out5, (2, 4, 0, 1, 3)).reshape(
        _BATCH, _SEQ, _NUM_FEATURES * _EMBED_DIM
    )
    return out


# speedup vs baseline: 5.3531x; 1.0116x over previous
import functools

import jax
import jax.numpy as jnp
from jax import lax
from jax.experimental import pallas as pl
from jax.experimental.pallas import tpu as pltpu
from jax.experimental.pallas import tpu_sc as plsc

_NUM_FEATURES = 26
_VOCAB = 100000
_EMBED_DIM = 16
_BATCH = 1024
_SEQ = 50

_NC = 2
_NS = 16
_NW = _NC * _NS
_NTASK = _NUM_FEATURES * _EMBED_DIM
_TPW = _NTASK // _NW
_SCHUNK = 5
_NCHUNK = _SEQ // _SCHUNK


@functools.lru_cache(maxsize=None)
def _make_kernel():
    mesh = plsc.VectorSubcoreMesh(core_axis_name="c", subcore_axis_name="s")

    @functools.partial(
        pl.kernel,
        mesh=mesh,
        out_type=jax.ShapeDtypeStruct(
            (_SEQ, _NUM_FEATURES * _EMBED_DIM // 8, _BATCH // 128, 8, 128),
            jnp.float32,
        ),
        scratch_types=[
            pltpu.VMEM((_VOCAB,), jnp.float32),
            pltpu.VMEM((2, _SCHUNK, _BATCH), jnp.int32),
            pltpu.VMEM((2, _SCHUNK, _BATCH // 128, 128), jnp.float32),
            pltpu.SemaphoreType.DMA,
            pltpu.SemaphoreType.DMA,
            pltpu.SemaphoreType.DMA,
        ],
        compiler_params=pltpu.CompilerParams(
            use_tc_tiling_on_sc=False, needs_layout_passes=False
        ),
    )
    def k(tab_hbm, feat_hbm, out_hbm, slab_v, idx_v, outb_v, slab_sem,
          idx_sem, out_sem):
        wid = lax.axis_index("s") * _NC + lax.axis_index("c")

        def task(j, carry):
            t = wid * _TPW + j
            i = t // _EMBED_DIM
            d = t % _EMBED_DIM
            c = i * _EMBED_DIM + d
            slab_cp = pltpu.async_copy(tab_hbm.at[i, d], slab_v, slab_sem)
            idx_cps = [None] * _NCHUNK
            out_cps = [None] * _NCHUNK
            for ch in (0, 1):
                idx_cps[ch] = pltpu.async_copy(
                    feat_hbm.at[i, pl.ds(ch * _SCHUNK, _SCHUNK)],
                    idx_v.at[ch], idx_sem,
                )
            slab_cp.wait()

            for ch in range(_NCHUNK):
                b = ch % 2
                idx_cps[ch].wait()
                if out_cps[ch - 2] is not None:
                    out_cps[ch - 2].wait()

                def srow(sl, carry3, _b=b):
                    @plsc.parallel_loop(0, _BATCH // 128, 1, unroll=2)
                    def grp(g):
                        for kk in range(8):
                            off = g * 128 + kk * 16
                            iv = idx_v.at[_b][sl, pl.ds(off, 16)]
                            vals = plsc.load_gather(slab_v, [iv])
                            outb_v.at[_b][sl, g, pl.ds(kk * 16, 16)] = vals

                    return carry3

                lax.fori_loop(0, _SCHUNK, srow, 0)
                out_cps[ch] = pltpu.async_copy(
                    outb_v.at[b],
                    out_hbm.at[pl.ds(ch * _SCHUNK, _SCHUNK), c // 8, :, c % 8],
                    out_sem,
                )
                if ch + 2 < _NCHUNK:
                    idx_cps[ch + 2] = pltpu.async_copy(
                        feat_hbm.at[i, pl.ds((ch + 2) * _SCHUNK, _SCHUNK)],
                        idx_v.at[b], idx_sem,
                    )
            out_cps[_NCHUNK - 2].wait()
            out_cps[_NCHUNK - 1].wait()
            return carry

        lax.fori_loop(0, _TPW, task, 0)

    return k


def kernel(features, tables):
    feat_t = jnp.transpose(features, (2, 1, 0))
    tab_t = jnp.transpose(tables, (0, 2, 1))
    out5 = _make_kernel()(tab_t, feat_t)
    out = jnp.transpose(out5, (2, 4, 0, 1, 3)).reshape(
        _BATCH, _SEQ, _NUM_FEATURES * _EMBED_DIM
    )
    return out
